# Initial kernel scaffold; baseline (speedup 1.0000x reference)
import jax
import jax.numpy as jnp
from jax.experimental import pallas as pl


def kernel(x, edge_index, batch, emb, W1, b1, W2, b2, W3, b3, Wp, bp):
    raise NotImplementedError("write your pallas kernel here")


# speedup vs baseline: 9.7651x; 9.7651x over previous
import functools

import jax
import jax.numpy as jnp
from jax import lax
from jax.experimental import pallas as pl
from jax.experimental.pallas import tpu as pltpu
from jax.experimental.pallas import tpu_sc as plsc

N = 50000
E = 800000
V = 10000
D = 64
H = 64
B = 512

NPAD = 50176
NROWCH = NPAD // 128
NECH = E // 128
BPAD = 640

_MESH = plsc.VectorSubcoreMesh(core_axis_name="c", subcore_axis_name="s")


def _fill_ones(ref):
    one = jnp.ones((16,), jnp.float32)

    def body(i, _):
        ref[pl.ds(i * 16, 16)] = one
        return 0

    lax.fori_loop(0, ref.shape[0] // 16, body, 0)


@functools.partial(
    pl.kernel,
    out_type=(
        jax.ShapeDtypeStruct((NPAD,), jnp.float32),
        jax.ShapeDtypeStruct((NPAD,), jnp.float32),
    ),
    mesh=_MESH,
    compiler_params=pltpu.CompilerParams(use_tc_tiling_on_sc=False),
    scratch_types=[
        pltpu.VMEM((128,), jnp.int32),
        pltpu.VMEM((128,), jnp.float32),
        pltpu.VMEM((128,), jnp.float32),
        pltpu.VMEM((128,), jnp.float32),
        pltpu.VMEM_SHARED((NPAD,), jnp.float32),
    ],
)
def _deg_kernel(dst_hbm, zc_hbm, deg0_hbm, deg1_hbm,
                idx_v, ones_v, zeros_v, buf_v, deg_sp):
    cid = lax.axis_index("c")
    sid = lax.axis_index("s")
    _fill_ones(ones_v)
    pltpu.sync_copy(zc_hbm, zeros_v)

    def init(i, _):
        g = i * 16 + sid

        @pl.when(g < NROWCH)
        def _():
            pltpu.sync_copy(zeros_v, deg_sp.at[pl.ds(g * 128, 128)])

        return 0

    lax.fori_loop(0, (NROWCH + 15) // 16, init, 0)
    plsc.subcore_barrier()

    half = NECH // 2

    def body(i, _):
        g = i * 16 + sid

        @pl.when(g < half)
        def _():
            esl = pl.ds((cid * half + g) * 128, 128)
            pltpu.sync_copy(dst_hbm.at[esl], idx_v)
            pltpu.sync_copy(ones_v, deg_sp.at[idx_v], add=True)

        return 0

    lax.fori_loop(0, (half + 15) // 16, body, 0)
    plsc.subcore_barrier()

    def drain(i, _):
        g = i * 16 + sid

        @pl.when(g < NROWCH)
        def _():
            sl = pl.ds(g * 128, 128)
            pltpu.sync_copy(deg_sp.at[sl], buf_v)

            @pl.when(cid == 0)
            def _():
                pltpu.sync_copy(buf_v, deg0_hbm.at[sl])

            @pl.when(cid == 1)
            def _():
                pltpu.sync_copy(buf_v, deg1_hbm.at[sl])

        return 0

    lax.fori_loop(0, (NROWCH + 15) // 16, drain, 0)


@functools.partial(
    pl.kernel,
    out_type=(
        jax.ShapeDtypeStruct((NPAD, 32), jnp.float32),
        jax.ShapeDtypeStruct((NPAD, 32), jnp.float32),
    ),
    mesh=_MESH,
    compiler_params=pltpu.CompilerParams(use_tc_tiling_on_sc=False),
    scratch_types=[
        pltpu.VMEM((128,), jnp.int32),
        pltpu.VMEM((128, 32), jnp.float32),
    ],
)
def _gather_kernel(x_hbm, t0_hbm, t1_hbm, g0_hbm, g1_hbm, idx_v, rows_v):
    cid = lax.axis_index("c")
    sid = lax.axis_index("s")

    def body(i, _):
        g = i * 16 + sid

        @pl.when(g < NROWCH)
        def _():
            sl = pl.ds(g * 128, 128)
            pltpu.sync_copy(x_hbm.at[sl], idx_v)

            @pl.when(cid == 0)
            def _():
                pltpu.sync_copy(t0_hbm.at[idx_v], rows_v)
                pltpu.sync_copy(rows_v, g0_hbm.at[sl])

            @pl.when(cid == 1)
            def _():
                pltpu.sync_copy(t1_hbm.at[idx_v], rows_v)
                pltpu.sync_copy(rows_v, g1_hbm.at[sl])

        return 0

    lax.fori_loop(0, (NROWCH + 15) // 16, body, 0)


@functools.partial(
    pl.kernel,
    out_type=(
        jax.ShapeDtypeStruct((NPAD, 32), jnp.float32),
        jax.ShapeDtypeStruct((NPAD, 32), jnp.float32),
    ),
    mesh=_MESH,
    compiler_params=pltpu.CompilerParams(use_tc_tiling_on_sc=False),
    scratch_types=[
        pltpu.VMEM((128,), jnp.int32),
        pltpu.VMEM((128,), jnp.int32),
        pltpu.VMEM((128, 32), jnp.float32),
        pltpu.VMEM_SHARED((NPAD, 32), jnp.float32),
    ],
)
def _prop_kernel(y0_hbm, y1_hbm, src_hbm, dst_hbm, o0_hbm, o1_hbm,
                 sidx_v, didx_v, rows_v, acc_sp):
    cid = lax.axis_index("c")
    sid = lax.axis_index("s")

    def init(i, _):
        g = i * 16 + sid

        @pl.when(g < NROWCH)
        def _():
            sl = pl.ds(g * 128, 128)

            @pl.when(cid == 0)
            def _():
                pltpu.sync_copy(y0_hbm.at[sl], rows_v)

            @pl.when(cid == 1)
            def _():
                pltpu.sync_copy(y1_hbm.at[sl], rows_v)

            pltpu.sync_copy(rows_v, acc_sp.at[sl])

        return 0

    lax.fori_loop(0, (NROWCH + 15) // 16, init, 0)
    plsc.subcore_barrier()

    def body(i, _):
        g = i * 16 + sid

        @pl.when(g < NECH)
        def _():
            esl = pl.ds(g * 128, 128)
            pltpu.sync_copy(src_hbm.at[esl], sidx_v)
            pltpu.sync_copy(dst_hbm.at[esl], didx_v)

            @pl.when(cid == 0)
            def _():
                pltpu.sync_copy(y0_hbm.at[sidx_v], rows_v)

            @pl.when(cid == 1)
            def _():
                pltpu.sync_copy(y1_hbm.at[sidx_v], rows_v)

            pltpu.sync_copy(rows_v, acc_sp.at[didx_v], add=True)

        return 0

    lax.fori_loop(0, (NECH + 15) // 16, body, 0)
    plsc.subcore_barrier()

    def drain(i, _):
        g = i * 16 + sid

        @pl.when(g < NROWCH)
        def _():
            sl = pl.ds(g * 128, 128)
            pltpu.sync_copy(acc_sp.at[sl], rows_v)

            @pl.when(cid == 0)
            def _():
                pltpu.sync_copy(rows_v, o0_hbm.at[sl])

            @pl.when(cid == 1)
            def _():
                pltpu.sync_copy(rows_v, o1_hbm.at[sl])

        return 0

    lax.fori_loop(0, (NROWCH + 15) // 16, drain, 0)


@functools.partial(
    pl.kernel,
    out_type=(
        jax.ShapeDtypeStruct((B, 32), jnp.float32),
        jax.ShapeDtypeStruct((B, 32), jnp.float32),
        jax.ShapeDtypeStruct((B,), jnp.float32),
    ),
    mesh=_MESH,
    compiler_params=pltpu.CompilerParams(use_tc_tiling_on_sc=False),
    scratch_types=[
        pltpu.VMEM((128,), jnp.int32),
        pltpu.VMEM((128, 32), jnp.float32),
        pltpu.VMEM((128,), jnp.float32),
        pltpu.VMEM((128,), jnp.float32),
        pltpu.VMEM((128, 32), jnp.float32),
        pltpu.VMEM_SHARED((BPAD, 32), jnp.float32),
        pltpu.VMEM_SHARED((BPAD,), jnp.float32),
    ],
)
def _pool_kernel(h0_hbm, h1_hbm, batch_hbm, zp_hbm, zc_hbm,
                 p0_hbm, p1_hbm, cnt_hbm,
                 bidx_v, rows_v, ones_v, zeros_v, zrow_v, pool_sp, cnt_sp):
    cid = lax.axis_index("c")
    sid = lax.axis_index("s")
    _fill_ones(ones_v)

    @pl.when(sid == 0)
    def _():
        pltpu.sync_copy(zp_hbm, zrow_v)
        pltpu.sync_copy(zc_hbm, zeros_v)
        for j in range(BPAD // 128):
            pltpu.sync_copy(zrow_v, pool_sp.at[pl.ds(j * 128, 128)])
            pltpu.sync_copy(zeros_v, cnt_sp.at[pl.ds(j * 128, 128)])

    plsc.subcore_barrier()

    def body(i, _):
        g = i * 16 + sid

        @pl.when(g < NROWCH)
        def _():
            sl = pl.ds(g * 128, 128)
            pltpu.sync_copy(batch_hbm.at[sl], bidx_v)

            @pl.when(cid == 0)
            def _():
                pltpu.sync_copy(h0_hbm.at[sl], rows_v)

            @pl.when(cid == 1)
            def _():
                pltpu.sync_copy(h1_hbm.at[sl], rows_v)

            pltpu.sync_copy(rows_v, pool_sp.at[bidx_v], add=True)

            @pl.when(cid == 0)
            def _():
                pltpu.sync_copy(ones_v, cnt_sp.at[bidx_v], add=True)

        return 0

    lax.fori_loop(0, (NROWCH + 15) // 16, body, 0)
    plsc.subcore_barrier()

    @pl.when(sid < 4)
    def _():
        sl = pl.ds(sid * 128, 128)
        pltpu.sync_copy(pool_sp.at[sl], rows_v)

        @pl.when(cid == 0)
        def _():
            pltpu.sync_copy(rows_v, p0_hbm.at[sl])
            pltpu.sync_copy(cnt_sp.at[sl], zeros_v)
            pltpu.sync_copy(zeros_v, cnt_hbm.at[sl])

        @pl.when(cid == 1)
        def _():
            pltpu.sync_copy(rows_v, p1_hbm.at[sl])


def _dinv_body(d0_ref, d1_ref, o_ref):
    o_ref[...] = lax.rsqrt(d0_ref[...] + d1_ref[...] + 1.0)


def _embw_body(emb_ref, w_ref, o0_ref, o1_ref):
    y = jnp.dot(emb_ref[...], w_ref[...], preferred_element_type=jnp.float32)
    o0_ref[...] = y[:, :32]
    o1_ref[...] = y[:, 32:]


def _scale_body(g0_ref, g1_ref, dv_ref, o0_ref, o1_ref):
    d = dv_ref[...]
    o0_ref[...] = d * g0_ref[...]
    o1_ref[...] = d * g1_ref[...]


def _mid_body(a0_ref, a1_ref, dv_ref, b_ref, w_ref, o0_ref, o1_ref):
    d = dv_ref[...]
    u = jnp.concatenate([a0_ref[...], a1_ref[...]], axis=1)
    h = jax.nn.relu(d * u + b_ref[...])
    y = jnp.dot(d * h, w_ref[...], preferred_element_type=jnp.float32)
    o0_ref[...] = y[:, :32]
    o1_ref[...] = y[:, 32:]


def _act_body(a0_ref, a1_ref, dv_ref, b_ref, o0_ref, o1_ref):
    d = dv_ref[...]
    b = b_ref[...]
    o0_ref[...] = jax.nn.relu(d * a0_ref[...] + b[:, :32])
    o1_ref[...] = jax.nn.relu(d * a1_ref[...] + b[:, 32:])


def _final_body(p0_ref, p1_ref, c_ref, w_ref, b_ref, o_ref):
    inv = 1.0 / jnp.maximum(c_ref[...], 1.0)
    u = jnp.concatenate([p0_ref[...], p1_ref[...]], axis=1) * inv
    o_ref[...] = (
        jnp.dot(u, w_ref[...], preferred_element_type=jnp.float32) + b_ref[...]
    )


_RBLK = 1024
_NBLK = NPAD // _RBLK


def _row_specs(n):
    return [pl.BlockSpec((_RBLK, 32), lambda i: (i, 0)) for _ in range(n)]


def _halves_out():
    return (
        jax.ShapeDtypeStruct((NPAD, 32), jnp.float32),
        jax.ShapeDtypeStruct((NPAD, 32), jnp.float32),
    )


def _mid_call(a0, a1, dinv2, b2, w):
    return pl.pallas_call(
        _mid_body,
        grid=(_NBLK,),
        in_specs=_row_specs(2)
        + [
            pl.BlockSpec((_RBLK, 1), lambda i: (i, 0)),
            pl.BlockSpec((1, 64), lambda i: (0, 0)),
            pl.BlockSpec((64, 64), lambda i: (0, 0)),
        ],
        out_specs=tuple(_row_specs(2)),
        out_shape=_halves_out(),
    )(a0, a1, dinv2, b2, w)


def _act_call(a0, a1, dinv2, b2):
    return pl.pallas_call(
        _act_body,
        grid=(_NBLK,),
        in_specs=_row_specs(2)
        + [
            pl.BlockSpec((_RBLK, 1), lambda i: (i, 0)),
            pl.BlockSpec((1, 64), lambda i: (0, 0)),
        ],
        out_specs=tuple(_row_specs(2)),
        out_shape=_halves_out(),
    )(a0, a1, dinv2, b2)


def kernel(x, edge_index, batch, emb, W1, b1, W2, b2, W3, b3, Wp, bp):
    x = x.astype(jnp.int32)
    src = edge_index[0]
    dst = edge_index[1]

    x_pad = jnp.concatenate([x, jnp.zeros((NPAD - N,), jnp.int32)])
    batch_pad = jnp.concatenate(
        [batch.astype(jnp.int32), jnp.full((NPAD - N,), B, jnp.int32)]
    )
    zerosP = jnp.zeros((128, 32), jnp.float32)
    zerosC = jnp.zeros((128,), jnp.float32)

    deg0, deg1 = _deg_kernel(dst, zerosC)
    dinv = pl.pallas_call(
        _dinv_body,
        out_shape=jax.ShapeDtypeStruct((NROWCH, 128), jnp.float32),
    )(deg0.reshape(NROWCH, 128), deg1.reshape(NROWCH, 128))
    dinv2 = dinv.reshape(NPAD, 1)

    embw0, embw1 = pl.pallas_call(
        _embw_body,
        grid=(5,),
        in_specs=[
            pl.BlockSpec((2000, 64), lambda i: (i, 0)),
            pl.BlockSpec((64, 64), lambda i: (0, 0)),
        ],
        out_specs=(
            pl.BlockSpec((2000, 32), lambda i: (i, 0)),
            pl.BlockSpec((2000, 32), lambda i: (i, 0)),
        ),
        out_shape=(
            jax.ShapeDtypeStruct((V, 32), jnp.float32),
            jax.ShapeDtypeStruct((V, 32), jnp.float32),
        ),
    )(emb, W1)
    g0, g1 = _gather_kernel(x_pad, embw0, embw1)

    y0, y1 = pl.pallas_call(
        _scale_body,
        grid=(_NBLK,),
        in_specs=_row_specs(2) + [pl.BlockSpec((_RBLK, 1), lambda i: (i, 0))],
        out_specs=tuple(_row_specs(2)),
        out_shape=_halves_out(),
    )(g0, g1, dinv2)

    a0, a1 = _prop_kernel(y0, y1, src, dst)
    y0, y1 = _mid_call(a0, a1, dinv2, b1[None, :], W2)
    a0, a1 = _prop_kernel(y0, y1, src, dst)
    y0, y1 = _mid_call(a0, a1, dinv2, b2[None, :], W3)
    a0, a1 = _prop_kernel(y0, y1, src, dst)
    h0, h1 = _act_call(a0, a1, dinv2, b3[None, :])

    p0, p1, cnt = _pool_kernel(h0, h1, batch_pad, zerosP, zerosC)
    out = pl.pallas_call(
        _final_body,
        in_specs=[
            pl.BlockSpec((B, 32), lambda: (0, 0)),
            pl.BlockSpec((B, 32), lambda: (0, 0)),
            pl.BlockSpec((B, 1), lambda: (0, 0)),
            pl.BlockSpec((64, 64), lambda: (0, 0)),
            pl.BlockSpec((1, 64), lambda: (0, 0)),
        ],
        out_specs=pl.BlockSpec((B, 64), lambda: (0, 0)),
        out_shape=jax.ShapeDtypeStruct((B, 64), jnp.float32),
    )(p0, p1, cnt.reshape(B, 1), Wp, bp[None, :])
    return out


# speedup vs baseline: 18.1344x; 1.8571x over previous
import functools

import jax
import jax.numpy as jnp
from jax import lax
from jax.experimental import pallas as pl
from jax.experimental.pallas import tpu as pltpu
from jax.experimental.pallas import tpu_sc as plsc

N = 50000
E = 800000
V = 10000
D = 64
H = 64
B = 512

NPAD = 51200
NROWCH = NPAD // 128
EPAD = 802816
NECH = EPAD // 128
CH_PER_SUB = NECH // 16
BPAD = 640

_MESH = plsc.VectorSubcoreMesh(core_axis_name="c", subcore_axis_name="s")
_SC_PARAMS = pltpu.CompilerParams(use_tc_tiling_on_sc=False)


def _fill_ones(ref):
    one = jnp.ones((16,), jnp.float32)

    def body(i, _):
        ref[pl.ds(i * 16, 16)] = one
        return 0

    lax.fori_loop(0, ref.shape[0] // 16, body, 0)


@functools.partial(
    pl.kernel,
    out_type=(
        jax.ShapeDtypeStruct((NPAD,), jnp.float32),
        jax.ShapeDtypeStruct((NPAD,), jnp.float32),
    ),
    mesh=_MESH,
    compiler_params=_SC_PARAMS,
    scratch_types=[
        pltpu.VMEM((128,), jnp.int32),
        pltpu.VMEM((128,), jnp.float32),
        pltpu.VMEM((128,), jnp.float32),
        pltpu.VMEM((128,), jnp.float32),
        pltpu.VMEM_SHARED((NPAD,), jnp.float32),
    ],
)
def _deg_kernel(dst_hbm, zc_hbm, deg0_hbm, deg1_hbm,
                idx_v, ones_v, zeros_v, buf_v, deg_sp):
    cid = lax.axis_index("c")
    sid = lax.axis_index("s")
    _fill_ones(ones_v)
    pltpu.sync_copy(zc_hbm, zeros_v)

    def init(i, _):
        g = i * 16 + sid
        pltpu.sync_copy(zeros_v, deg_sp.at[pl.ds(g * 128, 128)])
        return 0

    lax.fori_loop(0, NROWCH // 16, init, 0)
    plsc.subcore_barrier()

    half = NECH // 2

    def body(i, _):
        g = cid * half + i * 16 + sid
        pltpu.sync_copy(dst_hbm.at[pl.ds(g * 128, 128)], idx_v)
        pltpu.sync_copy(ones_v, deg_sp.at[idx_v], add=True)
        return 0

    lax.fori_loop(0, half // 16, body, 0)
    plsc.subcore_barrier()

    def drain(i, _):
        g = i * 16 + sid
        sl = pl.ds(g * 128, 128)
        pltpu.sync_copy(deg_sp.at[sl], buf_v)

        @pl.when(cid == 0)
        def _():
            pltpu.sync_copy(buf_v, deg0_hbm.at[sl])

        @pl.when(cid == 1)
        def _():
            pltpu.sync_copy(buf_v, deg1_hbm.at[sl])

        return 0

    lax.fori_loop(0, NROWCH // 16, drain, 0)


@functools.partial(
    pl.kernel,
    out_type=(
        jax.ShapeDtypeStruct((NPAD, 32), jnp.float32),
        jax.ShapeDtypeStruct((NPAD, 32), jnp.float32),
    ),
    mesh=_MESH,
    compiler_params=_SC_PARAMS,
    scratch_types=[
        pltpu.VMEM((128,), jnp.int32),
        pltpu.VMEM((128, 32), jnp.float32),
    ],
)
def _gather_kernel(x_hbm, t0_hbm, t1_hbm, g0_hbm, g1_hbm, idx_v, rows_v):
    cid = lax.axis_index("c")
    sid = lax.axis_index("s")

    def body(i, _):
        g = i * 16 + sid
        sl = pl.ds(g * 128, 128)
        pltpu.sync_copy(x_hbm.at[sl], idx_v)

        @pl.when(cid == 0)
        def _():
            pltpu.sync_copy(t0_hbm.at[idx_v], rows_v)
            pltpu.sync_copy(rows_v, g0_hbm.at[sl])

        @pl.when(cid == 1)
        def _():
            pltpu.sync_copy(t1_hbm.at[idx_v], rows_v)
            pltpu.sync_copy(rows_v, g1_hbm.at[sl])

        return 0

    lax.fori_loop(0, NROWCH // 16, body, 0)


_INIT_ROWS = 400


@functools.partial(
    pl.kernel,
    out_type=(
        jax.ShapeDtypeStruct((NPAD, 32), jnp.float32),
        jax.ShapeDtypeStruct((NPAD, 32), jnp.float32),
    ),
    mesh=_MESH,
    compiler_params=_SC_PARAMS,
    scratch_types=[
        pltpu.VMEM((8, 128), jnp.int32),
        pltpu.VMEM((8, 128), jnp.int32),
        pltpu.VMEM((128, 32), jnp.float32),
        pltpu.VMEM((128, 32), jnp.float32),
        pltpu.VMEM((_INIT_ROWS, 32), jnp.float32),
        pltpu.SemaphoreType.DMA,
        pltpu.SemaphoreType.DMA,
        pltpu.VMEM_SHARED((NPAD, 32), jnp.float32),
    ],
)
def _prop_kernel(y0_hbm, y1_hbm, src2_hbm, dst2_hbm, o0_hbm, o1_hbm,
                 sidx8_v, didx8_v, rows0_v, rows1_v, big_v,
                 sem0, sem1, acc_sp):
    cid = lax.axis_index("c")
    sid = lax.axis_index("s")
    rows = (rows0_v, rows1_v)
    sems = (sem0, sem1)

    def init(k, _):
        r0 = sid * 3200 + k * _INIT_ROWS
        sl = pl.ds(r0, _INIT_ROWS)

        @pl.when(cid == 0)
        def _():
            pltpu.sync_copy(y0_hbm.at[sl], big_v)

        @pl.when(cid == 1)
        def _():
            pltpu.sync_copy(y1_hbm.at[sl], big_v)

        pltpu.sync_copy(big_v, acc_sp.at[sl])
        return 0

    lax.fori_loop(0, 3200 // _INIT_ROWS, init, 0)
    plsc.subcore_barrier()

    def _edge_block(y_hbm):
        prev = None
        for j in range(8):
            cp = pltpu.async_copy(
                y_hbm.at[sidx8_v.at[j]], rows[j % 2], sems[j % 2]
            )
            if prev is not None:
                prev[0].wait()
                pltpu.sync_copy(
                    rows[(j - 1) % 2], acc_sp.at[didx8_v.at[j - 1]], add=True
                )
            prev = (cp,)
        prev[0].wait()
        pltpu.sync_copy(rows[7 % 2], acc_sp.at[didx8_v.at[7]], add=True)

    def body(b, _):
        r0 = sid * CH_PER_SUB + b * 8
        bsl = pl.ds(r0, 8)
        pltpu.sync_copy(src2_hbm.at[bsl], sidx8_v)
        pltpu.sync_copy(dst2_hbm.at[bsl], didx8_v)

        @pl.when(cid == 0)
        def _():
            _edge_block(y0_hbm)

        @pl.when(cid == 1)
        def _():
            _edge_block(y1_hbm)

        return 0

    lax.fori_loop(0, CH_PER_SUB // 8, body, 0)
    plsc.subcore_barrier()

    def drain(k, _):
        r0 = sid * 3200 + k * _INIT_ROWS
        sl = pl.ds(r0, _INIT_ROWS)
        pltpu.sync_copy(acc_sp.at[sl], big_v)

        @pl.when(cid == 0)
        def _():
            pltpu.sync_copy(big_v, o0_hbm.at[sl])

        @pl.when(cid == 1)
        def _():
            pltpu.sync_copy(big_v, o1_hbm.at[sl])

        return 0

    lax.fori_loop(0, 3200 // _INIT_ROWS, drain, 0)


@functools.partial(
    pl.kernel,
    out_type=(
        jax.ShapeDtypeStruct((B, 32), jnp.float32),
        jax.ShapeDtypeStruct((B, 32), jnp.float32),
        jax.ShapeDtypeStruct((B,), jnp.float32),
    ),
    mesh=_MESH,
    compiler_params=_SC_PARAMS,
    scratch_types=[
        pltpu.VMEM((128,), jnp.int32),
        pltpu.VMEM((128, 32), jnp.float32),
        pltpu.VMEM((128,), jnp.float32),
        pltpu.VMEM((128,), jnp.float32),
        pltpu.VMEM((128, 32), jnp.float32),
        pltpu.VMEM_SHARED((BPAD, 32), jnp.float32),
        pltpu.VMEM_SHARED((BPAD,), jnp.float32),
    ],
)
def _pool_kernel(h0_hbm, h1_hbm, batch_hbm, zp_hbm, zc_hbm,
                 p0_hbm, p1_hbm, cnt_hbm,
                 bidx_v, rows_v, ones_v, zeros_v, zrow_v, pool_sp, cnt_sp):
    cid = lax.axis_index("c")
    sid = lax.axis_index("s")
    _fill_ones(ones_v)

    @pl.when(sid == 0)
    def _():
        pltpu.sync_copy(zp_hbm, zrow_v)
        pltpu.sync_copy(zc_hbm, zeros_v)
        for j in range(BPAD // 128):
            pltpu.sync_copy(zrow_v, pool_sp.at[pl.ds(j * 128, 128)])
            pltpu.sync_copy(zeros_v, cnt_sp.at[pl.ds(j * 128, 128)])

    plsc.subcore_barrier()

    def body(i, _):
        g = i * 16 + sid
        sl = pl.ds(g * 128, 128)
        pltpu.sync_copy(batch_hbm.at[sl], bidx_v)

        @pl.when(cid == 0)
        def _():
            pltpu.sync_copy(h0_hbm.at[sl], rows_v)

        @pl.when(cid == 1)
        def _():
            pltpu.sync_copy(h1_hbm.at[sl], rows_v)

        pltpu.sync_copy(rows_v, pool_sp.at[bidx_v], add=True)

        @pl.when(cid == 0)
        def _():
            pltpu.sync_copy(ones_v, cnt_sp.at[bidx_v], add=True)

        return 0

    lax.fori_loop(0, NROWCH // 16, body, 0)
    plsc.subcore_barrier()

    @pl.when(sid < 4)
    def _():
        sl = pl.ds(sid * 128, 128)
        pltpu.sync_copy(pool_sp.at[sl], rows_v)

        @pl.when(cid == 0)
        def _():
            pltpu.sync_copy(rows_v, p0_hbm.at[sl])
            pltpu.sync_copy(cnt_sp.at[sl], zeros_v)
            pltpu.sync_copy(zeros_v, cnt_hbm.at[sl])

        @pl.when(cid == 1)
        def _():
            pltpu.sync_copy(rows_v, p1_hbm.at[sl])


def _dinv_body(d0_ref, d1_ref, o_ref):
    o_ref[...] = lax.rsqrt(d0_ref[...] + d1_ref[...] + 1.0)


def _embw_body(emb_ref, w_ref, o0_ref, o1_ref):
    y = jnp.dot(emb_ref[...], w_ref[...], preferred_element_type=jnp.float32)
    o0_ref[...] = y[:, :32]
    o1_ref[...] = y[:, 32:]


def _scale_body(g0_ref, g1_ref, dv_ref, o0_ref, o1_ref):
    d = dv_ref[...]
    o0_ref[...] = d * g0_ref[...]
    o1_ref[...] = d * g1_ref[...]


def _mid_body(a0_ref, a1_ref, dv_ref, b_ref, w_ref, o0_ref, o1_ref):
    d = dv_ref[...]
    u = jnp.concatenate([a0_ref[...], a1_ref[...]], axis=1)
    h = jax.nn.relu(d * u + b_ref[...])
    y = jnp.dot(d * h, w_ref[...], preferred_element_type=jnp.float32)
    o0_ref[...] = y[:, :32]
    o1_ref[...] = y[:, 32:]


def _act_body(a0_ref, a1_ref, dv_ref, b_ref, o0_ref, o1_ref):
    d = dv_ref[...]
    b = b_ref[...]
    o0_ref[...] = jax.nn.relu(d * a0_ref[...] + b[:, :32])
    o1_ref[...] = jax.nn.relu(d * a1_ref[...] + b[:, 32:])


def _final_body(p0_ref, p1_ref, c_ref, w_ref, b_ref, o_ref):
    inv = 1.0 / jnp.maximum(c_ref[...], 1.0)
    u = jnp.concatenate([p0_ref[...], p1_ref[...]], axis=1) * inv
    o_ref[...] = (
        jnp.dot(u, w_ref[...], preferred_element_type=jnp.float32) + b_ref[...]
    )


_RBLK = 1024
_NBLK = NPAD // _RBLK


def _row_specs(n):
    return [pl.BlockSpec((_RBLK, 32), lambda i: (i, 0)) for _ in range(n)]


def _halves_out():
    return (
        jax.ShapeDtypeStruct((NPAD, 32), jnp.float32),
        jax.ShapeDtypeStruct((NPAD, 32), jnp.float32),
    )


def _mid_call(a0, a1, dinv2, b2, w):
    return pl.pallas_call(
        _mid_body,
        grid=(_NBLK,),
        in_specs=_row_specs(2)
        + [
            pl.BlockSpec((_RBLK, 1), lambda i: (i, 0)),
            pl.BlockSpec((1, 64), lambda i: (0, 0)),
            pl.BlockSpec((64, 64), lambda i: (0, 0)),
        ],
        out_specs=tuple(_row_specs(2)),
        out_shape=_halves_out(),
    )(a0, a1, dinv2, b2, w)


def _act_call(a0, a1, dinv2, b2):
    return pl.pallas_call(
        _act_body,
        grid=(_NBLK,),
        in_specs=_row_specs(2)
        + [
            pl.BlockSpec((_RBLK, 1), lambda i: (i, 0)),
            pl.BlockSpec((1, 64), lambda i: (0, 0)),
        ],
        out_specs=tuple(_row_specs(2)),
        out_shape=_halves_out(),
    )(a0, a1, dinv2, b2)


def kernel(x, edge_index, batch, emb, W1, b1, W2, b2, W3, b3, Wp, bp):
    x = x.astype(jnp.int32)
    src = edge_index[0]
    dst = edge_index[1]

    pad_rows = (jnp.arange(EPAD - E, dtype=jnp.int32) % 1024) + N
    src_pad = jnp.concatenate([src, pad_rows])
    dst_pad = jnp.concatenate([dst, pad_rows])
    src2 = src_pad.reshape(NECH, 128)
    dst2 = dst_pad.reshape(NECH, 128)

    x_pad = jnp.concatenate([x, jnp.zeros((NPAD - N,), jnp.int32)])
    batch_pad = jnp.concatenate(
        [batch.astype(jnp.int32), jnp.full((NPAD - N,), B, jnp.int32)]
    )
    zerosP = jnp.zeros((128, 32), jnp.float32)
    zerosC = jnp.zeros((128,), jnp.float32)

    deg0, deg1 = _deg_kernel(dst_pad, zerosC)
    dinv = pl.pallas_call(
        _dinv_body,
        out_shape=jax.ShapeDtypeStruct((NROWCH, 128), jnp.float32),
    )(deg0.reshape(NROWCH, 128), deg1.reshape(NROWCH, 128))
    dinv2 = dinv.reshape(NPAD, 1)

    embw0, embw1 = pl.pallas_call(
        _embw_body,
        grid=(5,),
        in_specs=[
            pl.BlockSpec((2000, 64), lambda i: (i, 0)),
            pl.BlockSpec((64, 64), lambda i: (0, 0)),
        ],
        out_specs=(
            pl.BlockSpec((2000, 32), lambda i: (i, 0)),
            pl.BlockSpec((2000, 32), lambda i: (i, 0)),
        ),
        out_shape=(
            jax.ShapeDtypeStruct((V, 32), jnp.float32),
            jax.ShapeDtypeStruct((V, 32), jnp.float32),
        ),
    )(emb, W1)
    g0, g1 = _gather_kernel(x_pad, embw0, embw1)

    y0, y1 = pl.pallas_call(
        _scale_body,
        grid=(_NBLK,),
        in_specs=_row_specs(2) + [pl.BlockSpec((_RBLK, 1), lambda i: (i, 0))],
        out_specs=tuple(_row_specs(2)),
        out_shape=_halves_out(),
    )(g0, g1, dinv2)

    a0, a1 = _prop_kernel(y0, y1, src2, dst2)
    y0, y1 = _mid_call(a0, a1, dinv2, b1[None, :], W2)
    a0, a1 = _prop_kernel(y0, y1, src2, dst2)
    y0, y1 = _mid_call(a0, a1, dinv2, b2[None, :], W3)
    a0, a1 = _prop_kernel(y0, y1, src2, dst2)
    h0, h1 = _act_call(a0, a1, dinv2, b3[None, :])

    p0, p1, cnt = _pool_kernel(h0, h1, batch_pad, zerosP, zerosC)
    out = pl.pallas_call(
        _final_body,
        in_specs=[
            pl.BlockSpec((B, 32), lambda: (0, 0)),
            pl.BlockSpec((B, 32), lambda: (0, 0)),
            pl.BlockSpec((B, 1), lambda: (0, 0)),
            pl.BlockSpec((64, 64), lambda: (0, 0)),
            pl.BlockSpec((1, 64), lambda: (0, 0)),
        ],
        out_specs=pl.BlockSpec((B, 64), lambda: (0, 0)),
        out_shape=jax.ShapeDtypeStruct((B, 64), jnp.float32),
    )(p0, p1, cnt.reshape(B, 1), Wp, bp[None, :])
    return out


# speedup vs baseline: 19.9676x; 1.1011x over previous
import functools

import jax
import jax.numpy as jnp
from jax import lax
from jax.experimental import pallas as pl
from jax.experimental.pallas import tpu as pltpu
from jax.experimental.pallas import tpu_sc as plsc

N = 50000
E = 800000
V = 10000
D = 64
H = 64
B = 512

NPAD = 51200
NROWCH = NPAD // 128
EPAD = 802816
NECH = EPAD // 128
CH_PER_SUB = NECH // 16
BPAD = 640

_MESH = plsc.VectorSubcoreMesh(core_axis_name="c", subcore_axis_name="s")
_SC_PARAMS = pltpu.CompilerParams(use_tc_tiling_on_sc=False)


def _fill_ones(ref):
    one = jnp.ones((16,), jnp.float32)

    def body(i, _):
        ref[pl.ds(i * 16, 16)] = one
        return 0

    lax.fori_loop(0, ref.shape[0] // 16, body, 0)


@functools.partial(
    pl.kernel,
    out_type=(
        jax.ShapeDtypeStruct((NPAD,), jnp.float32),
        jax.ShapeDtypeStruct((NPAD,), jnp.float32),
    ),
    mesh=_MESH,
    compiler_params=_SC_PARAMS,
    scratch_types=[
        pltpu.VMEM((8, 128), jnp.int32),
        pltpu.VMEM((128,), jnp.float32),
        pltpu.VMEM((128,), jnp.float32),
        pltpu.VMEM((128,), jnp.float32),
        pltpu.SemaphoreType.DMA,
        pltpu.VMEM_SHARED((NPAD,), jnp.float32),
    ],
)
def _deg_kernel(dst2_hbm, zc_hbm, deg0_hbm, deg1_hbm,
                didx8_v, ones_v, zeros_v, buf_v, ssem, deg_sp):
    cid = lax.axis_index("c")
    sid = lax.axis_index("s")
    _fill_ones(ones_v)
    pltpu.sync_copy(zc_hbm, zeros_v)

    def init(i, _):
        g = i * 16 + sid
        pltpu.sync_copy(zeros_v, deg_sp.at[pl.ds(g * 128, 128)])
        return 0

    lax.fori_loop(0, NROWCH // 16, init, 0)
    plsc.subcore_barrier()

    halfb = NECH // 16

    def body(i, _):
        b = i * 16 + sid

        @pl.when(b < halfb)
        def _():
            pltpu.sync_copy(dst2_hbm.at[pl.ds((cid * halfb + b) * 8, 8)],
                            didx8_v)
            scp = [
                pltpu.async_copy(ones_v, deg_sp.at[didx8_v.at[j]], ssem,
                                 add=True)
                for j in range(8)
            ]
            for cp in scp:
                cp.wait()

        return 0

    lax.fori_loop(0, (halfb + 15) // 16, body, 0)
    plsc.subcore_barrier()

    def drain(i, _):
        g = i * 16 + sid
        sl = pl.ds(g * 128, 128)
        pltpu.sync_copy(deg_sp.at[sl], buf_v)

        @pl.when(cid == 0)
        def _():
            pltpu.sync_copy(buf_v, deg0_hbm.at[sl])

        @pl.when(cid == 1)
        def _():
            pltpu.sync_copy(buf_v, deg1_hbm.at[sl])

        return 0

    lax.fori_loop(0, NROWCH // 16, drain, 0)


@functools.partial(
    pl.kernel,
    out_type=(
        jax.ShapeDtypeStruct((NPAD, 32), jnp.float32),
        jax.ShapeDtypeStruct((NPAD, 32), jnp.float32),
    ),
    mesh=_MESH,
    compiler_params=_SC_PARAMS,
    scratch_types=[
        pltpu.VMEM((128,), jnp.int32),
        pltpu.VMEM((128, 32), jnp.float32),
    ],
)
def _gather_kernel(x_hbm, t0_hbm, t1_hbm, g0_hbm, g1_hbm, idx_v, rows_v):
    cid = lax.axis_index("c")
    sid = lax.axis_index("s")

    def body(i, _):
        g = i * 16 + sid
        sl = pl.ds(g * 128, 128)
        pltpu.sync_copy(x_hbm.at[sl], idx_v)

        @pl.when(cid == 0)
        def _():
            pltpu.sync_copy(t0_hbm.at[idx_v], rows_v)
            pltpu.sync_copy(rows_v, g0_hbm.at[sl])

        @pl.when(cid == 1)
        def _():
            pltpu.sync_copy(t1_hbm.at[idx_v], rows_v)
            pltpu.sync_copy(rows_v, g1_hbm.at[sl])

        return 0

    lax.fori_loop(0, NROWCH // 16, body, 0)


_INIT_ROWS = 200


@functools.partial(
    pl.kernel,
    out_type=(
        jax.ShapeDtypeStruct((NPAD, 32), jnp.float32),
        jax.ShapeDtypeStruct((NPAD, 32), jnp.float32),
    ),
    mesh=_MESH,
    compiler_params=_SC_PARAMS,
    scratch_types=[
        pltpu.VMEM((8, 128), jnp.int32),
        pltpu.VMEM((8, 128), jnp.int32),
        pltpu.VMEM((128, 32), jnp.float32),
        pltpu.VMEM((128, 32), jnp.float32),
        pltpu.VMEM((128, 32), jnp.float32),
        pltpu.VMEM((128, 32), jnp.float32),
        pltpu.VMEM((_INIT_ROWS, 32), jnp.float32),
        pltpu.SemaphoreType.DMA,
        pltpu.SemaphoreType.DMA,
        pltpu.SemaphoreType.DMA,
        pltpu.SemaphoreType.DMA,
        pltpu.SemaphoreType.DMA,
        pltpu.SemaphoreType.DMA,
        pltpu.SemaphoreType.DMA,
        pltpu.SemaphoreType.DMA,
        pltpu.VMEM_SHARED((NPAD, 32), jnp.float32),
    ],
)
def _prop_kernel(y0_hbm, y1_hbm, src2_hbm, dst2_hbm, o0_hbm, o1_hbm,
                 sidx8_v, didx8_v, rows0_v, rows1_v, rows2_v, rows3_v, big_v,
                 gs0, gs1, gs2, gs3, ss0, ss1, ss2, ss3, acc_sp):
    cid = lax.axis_index("c")
    sid = lax.axis_index("s")
    rows = (rows0_v, rows1_v, rows2_v, rows3_v)
    gsem = (gs0, gs1, gs2, gs3)
    ssem = (ss0, ss1, ss2, ss3)

    def init(k, _):
        r0 = sid * 3200 + k * _INIT_ROWS
        sl = pl.ds(r0, _INIT_ROWS)

        @pl.when(cid == 0)
        def _():
            pltpu.sync_copy(y0_hbm.at[sl], big_v)

        @pl.when(cid == 1)
        def _():
            pltpu.sync_copy(y1_hbm.at[sl], big_v)

        pltpu.sync_copy(big_v, acc_sp.at[sl])
        return 0

    lax.fori_loop(0, 3200 // _INIT_ROWS, init, 0)
    plsc.subcore_barrier()

    def _edge_block(y_hbm):
        gcp = [None] * 8
        scp = [None] * 8
        for j in range(8):
            if j >= 4:
                scp[j - 4].wait()
            gcp[j] = pltpu.async_copy(
                y_hbm.at[sidx8_v.at[j]], rows[j % 4], gsem[j % 4]
            )
            if j > 0:
                gcp[j - 1].wait()
                scp[j - 1] = pltpu.async_copy(
                    rows[(j - 1) % 4], acc_sp.at[didx8_v.at[j - 1]],
                    ssem[(j - 1) % 4], add=True,
                )
        gcp[7].wait()
        scp[7] = pltpu.async_copy(
            rows[3], acc_sp.at[didx8_v.at[7]], ssem[3], add=True
        )
        for j in range(4, 8):
            scp[j].wait()

    def body(b, _):
        r0 = sid * CH_PER_SUB + b * 8
        bsl = pl.ds(r0, 8)
        pltpu.sync_copy(src2_hbm.at[bsl], sidx8_v)
        pltpu.sync_copy(dst2_hbm.at[bsl], didx8_v)

        @pl.when(cid == 0)
        def _():
            _edge_block(y0_hbm)

        @pl.when(cid == 1)
        def _():
            _edge_block(y1_hbm)

        return 0

    lax.fori_loop(0, CH_PER_SUB // 8, body, 0)
    plsc.subcore_barrier()

    def drain(k, _):
        r0 = sid * 3200 + k * _INIT_ROWS
        sl = pl.ds(r0, _INIT_ROWS)
        pltpu.sync_copy(acc_sp.at[sl], big_v)

        @pl.when(cid == 0)
        def _():
            pltpu.sync_copy(big_v, o0_hbm.at[sl])

        @pl.when(cid == 1)
        def _():
            pltpu.sync_copy(big_v, o1_hbm.at[sl])

        return 0

    lax.fori_loop(0, 3200 // _INIT_ROWS, drain, 0)


@functools.partial(
    pl.kernel,
    out_type=(
        jax.ShapeDtypeStruct((B, 32), jnp.float32),
        jax.ShapeDtypeStruct((B, 32), jnp.float32),
        jax.ShapeDtypeStruct((B,), jnp.float32),
    ),
    mesh=_MESH,
    compiler_params=_SC_PARAMS,
    scratch_types=[
        pltpu.VMEM((128,), jnp.int32),
        pltpu.VMEM((128, 32), jnp.float32),
        pltpu.VMEM((128,), jnp.float32),
        pltpu.VMEM((128,), jnp.float32),
        pltpu.VMEM((128, 32), jnp.float32),
        pltpu.VMEM_SHARED((BPAD, 32), jnp.float32),
        pltpu.VMEM_SHARED((BPAD,), jnp.float32),
    ],
)
def _pool_kernel(h0_hbm, h1_hbm, batch_hbm, zp_hbm, zc_hbm,
                 p0_hbm, p1_hbm, cnt_hbm,
                 bidx_v, rows_v, ones_v, zeros_v, zrow_v, pool_sp, cnt_sp):
    cid = lax.axis_index("c")
    sid = lax.axis_index("s")
    _fill_ones(ones_v)

    @pl.when(sid == 0)
    def _():
        pltpu.sync_copy(zp_hbm, zrow_v)
        pltpu.sync_copy(zc_hbm, zeros_v)
        for j in range(BPAD // 128):
            pltpu.sync_copy(zrow_v, pool_sp.at[pl.ds(j * 128, 128)])
            pltpu.sync_copy(zeros_v, cnt_sp.at[pl.ds(j * 128, 128)])

    plsc.subcore_barrier()

    def body(i, _):
        g = i * 16 + sid
        sl = pl.ds(g * 128, 128)
        pltpu.sync_copy(batch_hbm.at[sl], bidx_v)

        @pl.when(cid == 0)
        def _():
            pltpu.sync_copy(h0_hbm.at[sl], rows_v)

        @pl.when(cid == 1)
        def _():
            pltpu.sync_copy(h1_hbm.at[sl], rows_v)

        pltpu.sync_copy(rows_v, pool_sp.at[bidx_v], add=True)

        @pl.when(cid == 0)
        def _():
            pltpu.sync_copy(ones_v, cnt_sp.at[bidx_v], add=True)

        return 0

    lax.fori_loop(0, NROWCH // 16, body, 0)
    plsc.subcore_barrier()

    @pl.when(sid < 4)
    def _():
        sl = pl.ds(sid * 128, 128)
        pltpu.sync_copy(pool_sp.at[sl], rows_v)

        @pl.when(cid == 0)
        def _():
            pltpu.sync_copy(rows_v, p0_hbm.at[sl])
            pltpu.sync_copy(cnt_sp.at[sl], zeros_v)
            pltpu.sync_copy(zeros_v, cnt_hbm.at[sl])

        @pl.when(cid == 1)
        def _():
            pltpu.sync_copy(rows_v, p1_hbm.at[sl])


def _dinv_body(d0_ref, d1_ref, o_ref):
    o_ref[...] = lax.rsqrt(d0_ref[...] + d1_ref[...] + 1.0)


def _embw_body(emb_ref, w_ref, o0_ref, o1_ref):
    y = jnp.dot(emb_ref[...], w_ref[...], preferred_element_type=jnp.float32)
    o0_ref[...] = y[:, :32]
    o1_ref[...] = y[:, 32:]


def _scale_body(g0_ref, g1_ref, dv_ref, o0_ref, o1_ref):
    d = dv_ref[...]
    o0_ref[...] = d * g0_ref[...]
    o1_ref[...] = d * g1_ref[...]


def _mid_body(a0_ref, a1_ref, dv_ref, b_ref, w_ref, o0_ref, o1_ref):
    d = dv_ref[...]
    u = jnp.concatenate([a0_ref[...], a1_ref[...]], axis=1)
    h = jax.nn.relu(d * u + b_ref[...])
    y = jnp.dot(d * h, w_ref[...], preferred_element_type=jnp.float32)
    o0_ref[...] = y[:, :32]
    o1_ref[...] = y[:, 32:]


def _act_body(a0_ref, a1_ref, dv_ref, b_ref, o0_ref, o1_ref):
    d = dv_ref[...]
    b = b_ref[...]
    o0_ref[...] = jax.nn.relu(d * a0_ref[...] + b[:, :32])
    o1_ref[...] = jax.nn.relu(d * a1_ref[...] + b[:, 32:])


def _final_body(p0_ref, p1_ref, c_ref, w_ref, b_ref, o_ref):
    inv = 1.0 / jnp.maximum(c_ref[...], 1.0)
    u = jnp.concatenate([p0_ref[...], p1_ref[...]], axis=1) * inv
    o_ref[...] = (
        jnp.dot(u, w_ref[...], preferred_element_type=jnp.float32) + b_ref[...]
    )


_RBLK = 1024
_NBLK = NPAD // _RBLK


def _row_specs(n):
    return [pl.BlockSpec((_RBLK, 32), lambda i: (i, 0)) for _ in range(n)]


def _halves_out():
    return (
        jax.ShapeDtypeStruct((NPAD, 32), jnp.float32),
        jax.ShapeDtypeStruct((NPAD, 32), jnp.float32),
    )


def _mid_call(a0, a1, dinv2, b2, w):
    return pl.pallas_call(
        _mid_body,
        grid=(_NBLK,),
        in_specs=_row_specs(2)
        + [
            pl.BlockSpec((_RBLK, 1), lambda i: (i, 0)),
            pl.BlockSpec((1, 64), lambda i: (0, 0)),
            pl.BlockSpec((64, 64), lambda i: (0, 0)),
        ],
        out_specs=tuple(_row_specs(2)),
        out_shape=_halves_out(),
    )(a0, a1, dinv2, b2, w)


def _act_call(a0, a1, dinv2, b2):
    return pl.pallas_call(
        _act_body,
        grid=(_NBLK,),
        in_specs=_row_specs(2)
        + [
            pl.BlockSpec((_RBLK, 1), lambda i: (i, 0)),
            pl.BlockSpec((1, 64), lambda i: (0, 0)),
        ],
        out_specs=tuple(_row_specs(2)),
        out_shape=_halves_out(),
    )(a0, a1, dinv2, b2)


def kernel(x, edge_index, batch, emb, W1, b1, W2, b2, W3, b3, Wp, bp):
    x = x.astype(jnp.int32)
    src = edge_index[0]
    dst = edge_index[1]

    pad_rows = (jnp.arange(EPAD - E, dtype=jnp.int32) % 1024) + N
    src_pad = jnp.concatenate([src, pad_rows])
    dst_pad = jnp.concatenate([dst, pad_rows])
    src2 = src_pad.reshape(NECH, 128)
    dst2 = dst_pad.reshape(NECH, 128)

    x_pad = jnp.concatenate([x, jnp.zeros((NPAD - N,), jnp.int32)])
    batch_pad = jnp.concatenate(
        [batch.astype(jnp.int32), jnp.full((NPAD - N,), B, jnp.int32)]
    )
    zerosP = jnp.zeros((128, 32), jnp.float32)
    zerosC = jnp.zeros((128,), jnp.float32)

    deg0, deg1 = _deg_kernel(dst2, zerosC)
    dinv = pl.pallas_call(
        _dinv_body,
        out_shape=jax.ShapeDtypeStruct((NROWCH, 128), jnp.float32),
    )(deg0.reshape(NROWCH, 128), deg1.reshape(NROWCH, 128))
    dinv2 = dinv.reshape(NPAD, 1)

    embw0, embw1 = pl.pallas_call(
        _embw_body,
        grid=(5,),
        in_specs=[
            pl.BlockSpec((2000, 64), lambda i: (i, 0)),
            pl.BlockSpec((64, 64), lambda i: (0, 0)),
        ],
        out_specs=(
            pl.BlockSpec((2000, 32), lambda i: (i, 0)),
            pl.BlockSpec((2000, 32), lambda i: (i, 0)),
        ),
        out_shape=(
            jax.ShapeDtypeStruct((V, 32), jnp.float32),
            jax.ShapeDtypeStruct((V, 32), jnp.float32),
        ),
    )(emb, W1)
    g0, g1 = _gather_kernel(x_pad, embw0, embw1)

    y0, y1 = pl.pallas_call(
        _scale_body,
        grid=(_NBLK,),
        in_specs=_row_specs(2) + [pl.BlockSpec((_RBLK, 1), lambda i: (i, 0))],
        out_specs=tuple(_row_specs(2)),
        out_shape=_halves_out(),
    )(g0, g1, dinv2)

    a0, a1 = _prop_kernel(y0, y1, src2, dst2)
    y0, y1 = _mid_call(a0, a1, dinv2, b1[None, :], W2)
    a0, a1 = _prop_kernel(y0, y1, src2, dst2)
    y0, y1 = _mid_call(a0, a1, dinv2, b2[None, :], W3)
    a0, a1 = _prop_kernel(y0, y1, src2, dst2)
    h0, h1 = _act_call(a0, a1, dinv2, b3[None, :])

    p0, p1, cnt = _pool_kernel(h0, h1, batch_pad, zerosP, zerosC)
    out = pl.pallas_call(
        _final_body,
        in_specs=[
            pl.BlockSpec((B, 32), lambda: (0, 0)),
            pl.BlockSpec((B, 32), lambda: (0, 0)),
            pl.BlockSpec((B, 1), lambda: (0, 0)),
            pl.BlockSpec((64, 64), lambda: (0, 0)),
            pl.BlockSpec((1, 64), lambda: (0, 0)),
        ],
        out_specs=pl.BlockSpec((B, 64), lambda: (0, 0)),
        out_shape=jax.ShapeDtypeStruct((B, 64), jnp.float32),
    )(p0, p1, cnt.reshape(B, 1), Wp, bp[None, :])
    return out


# speedup vs baseline: 20.4838x; 1.0258x over previous
import functools

import jax
import jax.numpy as jnp
from jax import lax
from jax.experimental import pallas as pl
from jax.experimental.pallas import tpu as pltpu
from jax.experimental.pallas import tpu_sc as plsc

N = 50000
E = 800000
V = 10000
D = 64
H = 64
B = 512

NPAD = 51200
NROWCH = NPAD // 128
EPAD = 802816
NECH = EPAD // 128
CH_PER_SUB = NECH // 16
BPAD = 640

_MESH = plsc.VectorSubcoreMesh(core_axis_name="c", subcore_axis_name="s")
_SC_PARAMS = pltpu.CompilerParams(use_tc_tiling_on_sc=False)


def _fill_ones(ref):
    one = jnp.ones((16,), jnp.float32)

    def body(i, _):
        ref[pl.ds(i * 16, 16)] = one
        return 0

    lax.fori_loop(0, ref.shape[0] // 16, body, 0)


@functools.partial(
    pl.kernel,
    out_type=(
        jax.ShapeDtypeStruct((NPAD,), jnp.float32),
        jax.ShapeDtypeStruct((NPAD,), jnp.float32),
    ),
    mesh=_MESH,
    compiler_params=_SC_PARAMS,
    scratch_types=[
        pltpu.VMEM((8, 128), jnp.int32),
        pltpu.VMEM((128,), jnp.float32),
        pltpu.VMEM((128,), jnp.float32),
        pltpu.VMEM((128,), jnp.float32),
        pltpu.SemaphoreType.DMA,
        pltpu.VMEM_SHARED((NPAD,), jnp.float32),
    ],
)
def _deg_kernel(dst2_hbm, zc_hbm, deg0_hbm, deg1_hbm,
                didx8_v, ones_v, zeros_v, buf_v, ssem, deg_sp):
    cid = lax.axis_index("c")
    sid = lax.axis_index("s")
    _fill_ones(ones_v)
    pltpu.sync_copy(zc_hbm, zeros_v)

    def init(i, _):
        g = i * 16 + sid
        pltpu.sync_copy(zeros_v, deg_sp.at[pl.ds(g * 128, 128)])
        return 0

    lax.fori_loop(0, NROWCH // 16, init, 0)
    plsc.subcore_barrier()

    halfb = NECH // 16

    def body(i, _):
        b = i * 16 + sid

        @pl.when(b < halfb)
        def _():
            pltpu.sync_copy(dst2_hbm.at[pl.ds((cid * halfb + b) * 8, 8)],
                            didx8_v)
            scp = [
                pltpu.async_copy(ones_v, deg_sp.at[didx8_v.at[j]], ssem,
                                 add=True)
                for j in range(8)
            ]
            for cp in scp:
                cp.wait()

        return 0

    lax.fori_loop(0, (halfb + 15) // 16, body, 0)
    plsc.subcore_barrier()

    def drain(i, _):
        g = i * 16 + sid
        sl = pl.ds(g * 128, 128)
        pltpu.sync_copy(deg_sp.at[sl], buf_v)

        @pl.when(cid == 0)
        def _():
            pltpu.sync_copy(buf_v, deg0_hbm.at[sl])

        @pl.when(cid == 1)
        def _():
            pltpu.sync_copy(buf_v, deg1_hbm.at[sl])

        return 0

    lax.fori_loop(0, NROWCH // 16, drain, 0)


@functools.partial(
    pl.kernel,
    out_type=(
        jax.ShapeDtypeStruct((NPAD, 32), jnp.float32),
        jax.ShapeDtypeStruct((NPAD, 32), jnp.float32),
    ),
    mesh=_MESH,
    compiler_params=_SC_PARAMS,
    scratch_types=[
        pltpu.VMEM((3200,), jnp.int32),
        pltpu.VMEM((128, 32), jnp.float32),
        pltpu.VMEM((128, 32), jnp.float32),
        pltpu.VMEM((128, 32), jnp.float32),
        pltpu.VMEM((128, 32), jnp.float32),
        pltpu.SemaphoreType.DMA,
        pltpu.SemaphoreType.DMA,
        pltpu.SemaphoreType.DMA,
        pltpu.SemaphoreType.DMA,
        pltpu.SemaphoreType.DMA,
        pltpu.SemaphoreType.DMA,
        pltpu.SemaphoreType.DMA,
        pltpu.SemaphoreType.DMA,
    ],
)
def _gather_kernel(x_hbm, t0_hbm, t1_hbm, g0_hbm, g1_hbm,
                   xall_v, r0, r1, r2, r3,
                   gs0, gs1, gs2, gs3, ss0, ss1, ss2, ss3):
    cid = lax.axis_index("c")
    sid = lax.axis_index("s")
    rows = (r0, r1, r2, r3)
    gsem = (gs0, gs1, gs2, gs3)
    ssem = (ss0, ss1, ss2, ss3)
    base = sid * 3200
    pltpu.sync_copy(x_hbm.at[pl.ds(base, 3200)], xall_v)

    def _run(t_hbm, g_hbm):
        nch = 3200 // 128
        gcp = [None] * nch
        scp = [None] * nch
        for j in range(nch):
            if j >= 4:
                scp[j - 4].wait()
            gcp[j] = pltpu.async_copy(
                t_hbm.at[xall_v.at[pl.ds(j * 128, 128)]],
                rows[j % 4], gsem[j % 4],
            )
            if j > 0:
                gcp[j - 1].wait()
                scp[j - 1] = pltpu.async_copy(
                    rows[(j - 1) % 4],
                    g_hbm.at[pl.ds(base + (j - 1) * 128, 128)],
                    ssem[(j - 1) % 4],
                )
        gcp[nch - 1].wait()
        scp[nch - 1] = pltpu.async_copy(
            rows[(nch - 1) % 4],
            g_hbm.at[pl.ds(base + (nch - 1) * 128, 128)],
            ssem[(nch - 1) % 4],
        )
        for j in range(nch - 4, nch):
            scp[j].wait()

    @pl.when(cid == 0)
    def _():
        _run(t0_hbm, g0_hbm)

    @pl.when(cid == 1)
    def _():
        _run(t1_hbm, g1_hbm)


_INIT_CH = 25


@functools.partial(
    pl.kernel,
    out_type=(
        jax.ShapeDtypeStruct((NPAD, 32), jnp.float32),
        jax.ShapeDtypeStruct((NPAD, 32), jnp.float32),
    ),
    mesh=_MESH,
    compiler_params=_SC_PARAMS,
    scratch_types=[
        pltpu.VMEM((8, 128), jnp.int32),
        pltpu.VMEM((8, 128), jnp.int32),
        pltpu.VMEM((128, 32), jnp.float32),
        pltpu.VMEM((128, 32), jnp.float32),
        pltpu.VMEM((128, 32), jnp.float32),
        pltpu.VMEM((128, 32), jnp.float32),
        pltpu.SemaphoreType.DMA,
        pltpu.SemaphoreType.DMA,
        pltpu.SemaphoreType.DMA,
        pltpu.SemaphoreType.DMA,
        pltpu.SemaphoreType.DMA,
        pltpu.SemaphoreType.DMA,
        pltpu.SemaphoreType.DMA,
        pltpu.SemaphoreType.DMA,
        pltpu.VMEM_SHARED((NPAD, 32), jnp.float32),
    ],
)
def _prop_kernel(y0_hbm, y1_hbm, src2_hbm, dst2_hbm, o0_hbm, o1_hbm,
                 sidx8_v, didx8_v, rows0_v, rows1_v, rows2_v, rows3_v,
                 gs0, gs1, gs2, gs3, ss0, ss1, ss2, ss3, acc_sp):
    cid = lax.axis_index("c")
    sid = lax.axis_index("s")
    rows = (rows0_v, rows1_v, rows2_v, rows3_v)
    gsem = (gs0, gs1, gs2, gs3)
    ssem = (ss0, ss1, ss2, ss3)

    def _init_stream(y_hbm):
        lcp = [None] * _INIT_CH
        for k in range(_INIT_CH):
            sl = pl.ds(sid * 3200 + k * 128, 128)
            lcp[k] = pltpu.async_copy(y_hbm.at[sl], rows[k % 2], gsem[k % 2])
            if k > 0:
                lcp[k - 1].wait()
                pltpu.sync_copy(
                    rows[(k - 1) % 2],
                    acc_sp.at[pl.ds(sid * 3200 + (k - 1) * 128, 128)],
                )
        lcp[_INIT_CH - 1].wait()
        pltpu.sync_copy(
            rows[(_INIT_CH - 1) % 2],
            acc_sp.at[pl.ds(sid * 3200 + (_INIT_CH - 1) * 128, 128)],
        )

    @pl.when(cid == 0)
    def _():
        _init_stream(y0_hbm)

    @pl.when(cid == 1)
    def _():
        _init_stream(y1_hbm)

    plsc.subcore_barrier()

    def _edge_block(y_hbm):
        gcp = [None] * 8
        scp = [None] * 8
        for j in range(8):
            if j >= 4:
                scp[j - 4].wait()
            gcp[j] = pltpu.async_copy(
                y_hbm.at[sidx8_v.at[j]], rows[j % 4], gsem[j % 4]
            )
            if j > 0:
                gcp[j - 1].wait()
                scp[j - 1] = pltpu.async_copy(
                    rows[(j - 1) % 4], acc_sp.at[didx8_v.at[j - 1]],
                    ssem[(j - 1) % 4], add=True,
                )
        gcp[7].wait()
        scp[7] = pltpu.async_copy(
            rows[3], acc_sp.at[didx8_v.at[7]], ssem[3], add=True
        )
        for j in range(4, 8):
            scp[j].wait()

    def body(b, _):
        r0 = sid * CH_PER_SUB + b * 8
        bsl = pl.ds(r0, 8)
        pltpu.sync_copy(src2_hbm.at[bsl], sidx8_v)
        pltpu.sync_copy(dst2_hbm.at[bsl], didx8_v)

        @pl.when(cid == 0)
        def _():
            _edge_block(y0_hbm)

        @pl.when(cid == 1)
        def _():
            _edge_block(y1_hbm)

        return 0

    lax.fori_loop(0, CH_PER_SUB // 8, body, 0)
    plsc.subcore_barrier()

    def _drain_stream(o_hbm):
        ocp = [None] * _INIT_CH
        for k in range(_INIT_CH):
            if k >= 2:
                ocp[k - 2].wait()
            sl = pl.ds(sid * 3200 + k * 128, 128)
            pltpu.sync_copy(acc_sp.at[sl], rows[k % 2])
            ocp[k] = pltpu.async_copy(rows[k % 2], o_hbm.at[sl], ssem[k % 2])
        ocp[_INIT_CH - 2].wait()
        ocp[_INIT_CH - 1].wait()

    @pl.when(cid == 0)
    def _():
        _drain_stream(o0_hbm)

    @pl.when(cid == 1)
    def _():
        _drain_stream(o1_hbm)


@functools.partial(
    pl.kernel,
    out_type=(
        jax.ShapeDtypeStruct((B, 32), jnp.float32),
        jax.ShapeDtypeStruct((B, 32), jnp.float32),
        jax.ShapeDtypeStruct((B,), jnp.float32),
    ),
    mesh=_MESH,
    compiler_params=_SC_PARAMS,
    scratch_types=[
        pltpu.VMEM((128,), jnp.int32),
        pltpu.VMEM((128, 32), jnp.float32),
        pltpu.VMEM((128,), jnp.float32),
        pltpu.VMEM((128,), jnp.float32),
        pltpu.VMEM((128, 32), jnp.float32),
        pltpu.VMEM_SHARED((BPAD, 32), jnp.float32),
        pltpu.VMEM_SHARED((BPAD,), jnp.float32),
    ],
)
def _pool_kernel(h0_hbm, h1_hbm, batch_hbm, zp_hbm, zc_hbm,
                 p0_hbm, p1_hbm, cnt_hbm,
                 bidx_v, rows_v, ones_v, zeros_v, zrow_v, pool_sp, cnt_sp):
    cid = lax.axis_index("c")
    sid = lax.axis_index("s")
    _fill_ones(ones_v)

    @pl.when(sid == 0)
    def _():
        pltpu.sync_copy(zp_hbm, zrow_v)
        pltpu.sync_copy(zc_hbm, zeros_v)
        for j in range(BPAD // 128):
            pltpu.sync_copy(zrow_v, pool_sp.at[pl.ds(j * 128, 128)])
            pltpu.sync_copy(zeros_v, cnt_sp.at[pl.ds(j * 128, 128)])

    plsc.subcore_barrier()

    def body(i, _):
        g = i * 16 + sid
        sl = pl.ds(g * 128, 128)
        pltpu.sync_copy(batch_hbm.at[sl], bidx_v)

        @pl.when(cid == 0)
        def _():
            pltpu.sync_copy(h0_hbm.at[sl], rows_v)

        @pl.when(cid == 1)
        def _():
            pltpu.sync_copy(h1_hbm.at[sl], rows_v)

        pltpu.sync_copy(rows_v, pool_sp.at[bidx_v], add=True)

        @pl.when(cid == 0)
        def _():
            pltpu.sync_copy(ones_v, cnt_sp.at[bidx_v], add=True)

        return 0

    lax.fori_loop(0, NROWCH // 16, body, 0)
    plsc.subcore_barrier()

    @pl.when(sid < 4)
    def _():
        sl = pl.ds(sid * 128, 128)
        pltpu.sync_copy(pool_sp.at[sl], rows_v)

        @pl.when(cid == 0)
        def _():
            pltpu.sync_copy(rows_v, p0_hbm.at[sl])
            pltpu.sync_copy(cnt_sp.at[sl], zeros_v)
            pltpu.sync_copy(zeros_v, cnt_hbm.at[sl])

        @pl.when(cid == 1)
        def _():
            pltpu.sync_copy(rows_v, p1_hbm.at[sl])


def _dinv_body(d0_ref, d1_ref, o_ref):
    o_ref[...] = lax.rsqrt(d0_ref[...] + d1_ref[...] + 1.0)


def _embw_body(emb_ref, w_ref, o0_ref, o1_ref):
    y = jnp.dot(emb_ref[...], w_ref[...], preferred_element_type=jnp.float32)
    o0_ref[...] = y[:, :32]
    o1_ref[...] = y[:, 32:]


def _scale_body(g0_ref, g1_ref, dv_ref, o0_ref, o1_ref):
    d = dv_ref[...]
    o0_ref[...] = d * g0_ref[...]
    o1_ref[...] = d * g1_ref[...]


def _mid_body(a0_ref, a1_ref, dv_ref, b_ref, w_ref, o0_ref, o1_ref):
    d = dv_ref[...]
    u = jnp.concatenate([a0_ref[...], a1_ref[...]], axis=1)
    h = jax.nn.relu(d * u + b_ref[...])
    y = jnp.dot(d * h, w_ref[...], preferred_element_type=jnp.float32)
    o0_ref[...] = y[:, :32]
    o1_ref[...] = y[:, 32:]


def _act_body(a0_ref, a1_ref, dv_ref, b_ref, o0_ref, o1_ref):
    d = dv_ref[...]
    b = b_ref[...]
    o0_ref[...] = jax.nn.relu(d * a0_ref[...] + b[:, :32])
    o1_ref[...] = jax.nn.relu(d * a1_ref[...] + b[:, 32:])


def _final_body(p0_ref, p1_ref, c_ref, w_ref, b_ref, o_ref):
    inv = 1.0 / jnp.maximum(c_ref[...], 1.0)
    u = jnp.concatenate([p0_ref[...], p1_ref[...]], axis=1) * inv
    o_ref[...] = (
        jnp.dot(u, w_ref[...], preferred_element_type=jnp.float32) + b_ref[...]
    )


_RBLK = 1024
_NBLK = NPAD // _RBLK


def _row_specs(n):
    return [pl.BlockSpec((_RBLK, 32), lambda i: (i, 0)) for _ in range(n)]


def _halves_out():
    return (
        jax.ShapeDtypeStruct((NPAD, 32), jnp.float32),
        jax.ShapeDtypeStruct((NPAD, 32), jnp.float32),
    )


def _mid_call(a0, a1, dinv2, b2, w):
    return pl.pallas_call(
        _mid_body,
        grid=(_NBLK,),
        in_specs=_row_specs(2)
        + [
            pl.BlockSpec((_RBLK, 1), lambda i: (i, 0)),
            pl.BlockSpec((1, 64), lambda i: (0, 0)),
            pl.BlockSpec((64, 64), lambda i: (0, 0)),
        ],
        out_specs=tuple(_row_specs(2)),
        out_shape=_halves_out(),
    )(a0, a1, dinv2, b2, w)


def _act_call(a0, a1, dinv2, b2):
    return pl.pallas_call(
        _act_body,
        grid=(_NBLK,),
        in_specs=_row_specs(2)
        + [
            pl.BlockSpec((_RBLK, 1), lambda i: (i, 0)),
            pl.BlockSpec((1, 64), lambda i: (0, 0)),
        ],
        out_specs=tuple(_row_specs(2)),
        out_shape=_halves_out(),
    )(a0, a1, dinv2, b2)


def kernel(x, edge_index, batch, emb, W1, b1, W2, b2, W3, b3, Wp, bp):
    x = x.astype(jnp.int32)
    src = edge_index[0]
    dst = edge_index[1]

    pad_rows = (jnp.arange(EPAD - E, dtype=jnp.int32) % 1024) + N
    src_pad = jnp.concatenate([src, pad_rows])
    dst_pad = jnp.concatenate([dst, pad_rows])
    src2 = src_pad.reshape(NECH, 128)
    dst2 = dst_pad.reshape(NECH, 128)

    x_pad = jnp.concatenate([x, jnp.zeros((NPAD - N,), jnp.int32)])
    batch_pad = jnp.concatenate(
        [batch.astype(jnp.int32), jnp.full((NPAD - N,), B, jnp.int32)]
    )
    zerosP = jnp.zeros((128, 32), jnp.float32)
    zerosC = jnp.zeros((128,), jnp.float32)

    deg0, deg1 = _deg_kernel(dst2, zerosC)
    dinv = pl.pallas_call(
        _dinv_body,
        out_shape=jax.ShapeDtypeStruct((NROWCH, 128), jnp.float32),
    )(deg0.reshape(NROWCH, 128), deg1.reshape(NROWCH, 128))
    dinv2 = dinv.reshape(NPAD, 1)

    embw0, embw1 = pl.pallas_call(
        _embw_body,
        grid=(5,),
        in_specs=[
            pl.BlockSpec((2000, 64), lambda i: (i, 0)),
            pl.BlockSpec((64, 64), lambda i: (0, 0)),
        ],
        out_specs=(
            pl.BlockSpec((2000, 32), lambda i: (i, 0)),
            pl.BlockSpec((2000, 32), lambda i: (i, 0)),
        ),
        out_shape=(
            jax.ShapeDtypeStruct((V, 32), jnp.float32),
            jax.ShapeDtypeStruct((V, 32), jnp.float32),
        ),
    )(emb, W1)
    g0, g1 = _gather_kernel(x_pad, embw0, embw1)

    y0, y1 = pl.pallas_call(
        _scale_body,
        grid=(_NBLK,),
        in_specs=_row_specs(2) + [pl.BlockSpec((_RBLK, 1), lambda i: (i, 0))],
        out_specs=tuple(_row_specs(2)),
        out_shape=_halves_out(),
    )(g0, g1, dinv2)

    a0, a1 = _prop_kernel(y0, y1, src2, dst2)
    y0, y1 = _mid_call(a0, a1, dinv2, b1[None, :], W2)
    a0, a1 = _prop_kernel(y0, y1, src2, dst2)
    y0, y1 = _mid_call(a0, a1, dinv2, b2[None, :], W3)
    a0, a1 = _prop_kernel(y0, y1, src2, dst2)
    h0, h1 = _act_call(a0, a1, dinv2, b3[None, :])

    p0, p1, cnt = _pool_kernel(h0, h1, batch_pad, zerosP, zerosC)
    out = pl.pallas_call(
        _final_body,
        in_specs=[
            pl.BlockSpec((B, 32), lambda: (0, 0)),
            pl.BlockSpec((B, 32), lambda: (0, 0)),
            pl.BlockSpec((B, 1), lambda: (0, 0)),
            pl.BlockSpec((64, 64), lambda: (0, 0)),
            pl.BlockSpec((1, 64), lambda: (0, 0)),
        ],
        out_specs=pl.BlockSpec((B, 64), lambda: (0, 0)),
        out_shape=jax.ShapeDtypeStruct((B, 64), jnp.float32),
    )(p0, p1, cnt.reshape(B, 1), Wp, bp[None, :])
    return out
